# baseline (speedup 1.0000x reference)
import functools

import jax
import jax.numpy as jnp
from jax import lax
from jax.experimental import pallas as pl
from jax.experimental.pallas import tpu as pltpu
from jax.experimental.pallas import tpu_sc as plsc

E = 320000
D = 128
N = 10000
NT = 16
Q = D // (2 * NT)
TW = N * Q
CE = 800
EH = E // 2
NCH = EH // CE
SW = EH // NT
CR = 2000

_mesh = plsc.VectorSubcoreMesh(core_axis_name="c", subcore_axis_name="s")


@functools.partial(
    pl.kernel,
    out_type=(
        jax.ShapeDtypeStruct((E,), jnp.float32),
        jax.ShapeDtypeStruct((NT * E,), jnp.float32),
    ),
    mesh=_mesh,
    scratch_types=[
        pltpu.VMEM((TW,), jnp.int32),
        pltpu.VMEM((TW,), jnp.int32),
        pltpu.VMEM((CE,), jnp.int32),
        pltpu.VMEM((CE,), jnp.int32),
        pltpu.VMEM((CE,), jnp.int32),
        pltpu.VMEM((CE,), jnp.int32),
        pltpu.VMEM((CE,), jnp.float32),
        pltpu.VMEM((CE,), jnp.float32),
        [pltpu.VMEM((CR,), jnp.float32)] * NT,
        pltpu.VMEM((CR,), jnp.float32),
        pltpu.SemaphoreType.DMA,
        pltpu.SemaphoreType.DMA,
        pltpu.SemaphoreType.DMA,
        pltpu.SemaphoreType.DMA,
        pltpu.SemaphoreType.DMA,
        pltpu.SemaphoreType.DMA,
    ],
    compiler_params=pltpu.CompilerParams(needs_layout_passes=False),
)
def _sddmm(src_idx_hbm, dst_idx_hbm, st_hbm, dt_hbm, out_hbm, par_hbm,
           st, dt, sidx0, sidx1, didx0, didx1, par0, par1, red, accv,
           sem_i0, sem_i1, sem_p0, sem_p1, sem_t, sem_r):
    sidx = (sidx0, sidx1)
    didx = (didx0, didx1)
    par = (par0, par1)
    sem_i = (sem_i0, sem_i1)
    sem_p = (sem_p0, sem_p1)
    cid = lax.axis_index("c")
    tid = lax.axis_index("s")
    half = cid * EH
    pbase = tid * E + half
    last = NCH - 1

    pltpu.async_copy(st_hbm.at[pl.ds(tid * TW, TW)], st, sem_t)
    pltpu.async_copy(dt_hbm.at[pl.ds(tid * TW, TW)], dt, sem_t)

    def fire_idx(b, c):
        off = half + jnp.minimum(c, last) * CE
        pltpu.async_copy(src_idx_hbm.at[pl.ds(off, CE)], sidx[b], sem_i[b])
        pltpu.async_copy(dst_idx_hbm.at[pl.ds(off, CE)], didx[b], sem_i[b])

    def wait_idx(b):
        pltpu.make_async_copy(
            src_idx_hbm.at[pl.ds(half, CE)], sidx[b], sem_i[b]).wait()
        pltpu.make_async_copy(
            dst_idx_hbm.at[pl.ds(half, CE)], didx[b], sem_i[b]).wait()

    def wait_par(b):
        pltpu.make_async_copy(
            par[b], par_hbm.at[pl.ds(pbase, CE)], sem_p[b]).wait()

    fire_idx(0, 0)
    fire_idx(1, 1)
    pltpu.make_async_copy(st_hbm.at[pl.ds(0, TW)], st, sem_t).wait()
    pltpu.make_async_copy(dt_hbm.at[pl.ds(0, TW)], dt, sem_t).wait()

    lane = lax.iota(jnp.int32, 16)
    himask = jnp.full((16,), -65536, jnp.int32)
    colrot = tuple(jnp.bitwise_and(lane + q, Q - 1) for q in range(Q))

    def s_body(s, carry):
        for b in range(2):
            c = s * 2 + b
            wait_idx(b)

            @pl.when(s > 0)
            def _():
                wait_par(b)

            def g_body(g, c2):
                sg = sidx[b][pl.ds(g * 16, 16)] * Q
                dg = didx[b][pl.ds(g * 16, 16)] * Q
                acc = jnp.zeros((16,), jnp.float32)
                for q in range(Q):
                    sv = plsc.load_gather(st, [sg + colrot[q]])
                    dv = plsc.load_gather(dt, [dg + colrot[q]])
                    s_lo = plsc.bitcast(lax.shift_left(sv, 16), jnp.float32)
                    d_lo = plsc.bitcast(lax.shift_left(dv, 16), jnp.float32)
                    s_hi = plsc.bitcast(
                        jnp.bitwise_and(sv, himask), jnp.float32)
                    d_hi = plsc.bitcast(
                        jnp.bitwise_and(dv, himask), jnp.float32)
                    acc = acc + s_lo * d_lo
                    acc = acc + s_hi * d_hi
                par[b][pl.ds(g * 16, 16)] = acc
                return c2

            lax.fori_loop(0, CE // 16, g_body, 0)
            pltpu.async_copy(
                par[b], par_hbm.at[pl.ds(pbase + c * CE, CE)], sem_p[b])
            fire_idx(b, c + 2)
        return carry

    lax.fori_loop(0, NCH // 2, s_body, 0)

    wait_idx(0)
    wait_idx(1)
    wait_par(0)
    wait_par(1)
    plsc.subcore_barrier()

    sbase = half + tid * SW
    for sub in range(SW // CR):
        off = sbase + sub * CR
        for t in range(NT):
            pltpu.async_copy(
                par_hbm.at[pl.ds(t * E + off, CR)], red[t], sem_r)
        for t in range(NT):
            pltpu.make_async_copy(
                par_hbm.at[pl.ds(0, CR)], red[t], sem_r).wait()

        def r_body(v, c2):
            acc = red[0][pl.ds(v * 16, 16)]
            for t in range(1, NT):
                acc = acc + red[t][pl.ds(v * 16, 16)]
            accv[pl.ds(v * 16, 16)] = acc
            return c2

        lax.fori_loop(0, CR // 16, r_body, 0)
        pltpu.sync_copy(accv, out_hbm.at[pl.ds(off, CR)])


def kernel(src_idx, dst_idx, src_feat, dst_feat):
    def shard(feat):
        pk = lax.bitcast_convert_type(
            feat.astype(jnp.bfloat16).reshape(N, D // 2, 2), jnp.int32)
        return pk.reshape(N, NT, Q).transpose(1, 0, 2).reshape(NT * TW)

    out, _ = _sddmm(src_idx.astype(jnp.int32), dst_idx.astype(jnp.int32),
                    shard(src_feat), shard(dst_feat))
    return out


# speedup vs baseline: 1.5863x; 1.5863x over previous
import functools

import jax
import jax.numpy as jnp
from jax import lax
from jax.experimental import pallas as pl
from jax.experimental.pallas import tpu as pltpu
from jax.experimental.pallas import tpu_sc as plsc

E = 320000
D = 128
N = 10000
NT = 16
Q = D // (2 * NT)
TW = N * Q
CE = 800
EH = E // 2
NCH = EH // CE
SW = EH // NT
CR = 2000
CN = 400

_mesh = plsc.VectorSubcoreMesh(core_axis_name="c", subcore_axis_name="s")


@functools.partial(
    pl.kernel,
    out_type=(
        jax.ShapeDtypeStruct((E,), jnp.float32),
        jax.ShapeDtypeStruct((NT * E,), jnp.float32),
    ),
    mesh=_mesh,
    scratch_types=[
        pltpu.VMEM((TW,), jnp.int32),
        pltpu.VMEM((TW,), jnp.int32),
        pltpu.VMEM((CE,), jnp.int32),
        pltpu.VMEM((CE,), jnp.int32),
        pltpu.VMEM((CE,), jnp.int32),
        pltpu.VMEM((CE,), jnp.int32),
        pltpu.VMEM((CE,), jnp.float32),
        pltpu.VMEM((CE,), jnp.float32),
        [pltpu.VMEM((CR,), jnp.float32)] * NT,
        pltpu.VMEM((CR,), jnp.float32),
        [pltpu.VMEM((CN,), jnp.float32)] * 8,
        pltpu.SemaphoreType.DMA,
        pltpu.SemaphoreType.DMA,
        pltpu.SemaphoreType.DMA,
        pltpu.SemaphoreType.DMA,
        pltpu.SemaphoreType.DMA,
        pltpu.SemaphoreType.DMA,
        pltpu.SemaphoreType.DMA,
    ],
    compiler_params=pltpu.CompilerParams(needs_layout_passes=False),
)
def _sddmm(src_idx_hbm, dst_idx_hbm, st_hbm, dt_hbm, out_hbm, par_hbm,
           st, dt, sidx0, sidx1, didx0, didx1, par0, par1, red, accv, stg,
           sem_i0, sem_i1, sem_p0, sem_p1, sem_t, sem_r, sem_s):
    sidx = (sidx0, sidx1)
    didx = (didx0, didx1)
    par = (par0, par1)
    sem_i = (sem_i0, sem_i1)
    sem_p = (sem_p0, sem_p1)
    cid = lax.axis_index("c")
    tid = lax.axis_index("s")
    half = cid * EH
    pbase = tid * E + half
    last = NCH - 1


    def fire_idx(b, c):
        off = half + jnp.minimum(c, last) * CE
        pltpu.async_copy(src_idx_hbm.at[pl.ds(off, CE)], sidx[b], sem_i[b])
        pltpu.async_copy(dst_idx_hbm.at[pl.ds(off, CE)], didx[b], sem_i[b])

    def wait_idx(b):
        pltpu.make_async_copy(
            src_idx_hbm.at[pl.ds(half, CE)], sidx[b], sem_i[b]).wait()
        pltpu.make_async_copy(
            dst_idx_hbm.at[pl.ds(half, CE)], didx[b], sem_i[b]).wait()

    def wait_par(b):
        pltpu.make_async_copy(
            par[b], par_hbm.at[pl.ds(pbase, CE)], sem_p[b]).wait()

    fire_idx(0, 0)
    fire_idx(1, 1)

    lane = lax.iota(jnp.int32, 16)
    lane4 = lane * 4
    himask = jnp.full((16,), -65536, jnp.int32)

    def load_slice(src_hbm, dst_ref):
        rbase = 8 * tid * N

        def c_body(cb, carry):
            nb = cb * CN
            for d8 in range(8):
                pltpu.async_copy(
                    src_hbm.at[pl.ds(rbase + d8 * N + nb, CN)],
                    stg[d8], sem_s)
            for d8 in range(8):
                pltpu.make_async_copy(
                    src_hbm.at[pl.ds(0, CN)], stg[d8], sem_s).wait()

            def rnd(v):
                b = plsc.bitcast(v, jnp.int32)
                return b + 32767 + jnp.bitwise_and(
                    lax.shift_right_logical(b, 16), 1)

            def n_body(g, c2):
                idx0 = (nb + g * 16) * 4 + lane4
                for q in range(Q):
                    ev = rnd(stg[2 * q][pl.ds(g * 16, 16)])
                    ov = rnd(stg[2 * q + 1][pl.ds(g * 16, 16)])
                    pk16 = jnp.bitwise_or(
                        lax.shift_right_logical(ev, 16),
                        jnp.bitwise_and(ov, himask))
                    plsc.store_scatter(dst_ref, [idx0 + q], pk16)
                return c2

            lax.fori_loop(0, CN // 16, n_body, 0)
            return carry

        lax.fori_loop(0, N // CN, c_body, 0)

    load_slice(st_hbm, st)
    load_slice(dt_hbm, dt)
    colrot = tuple(jnp.bitwise_and(lane + q, Q - 1) for q in range(Q))

    def s_body(s, carry):
        for b in range(2):
            c = s * 2 + b
            wait_idx(b)

            @pl.when(s > 0)
            def _():
                wait_par(b)

            def g_body(g, c2):
                sg = sidx[b][pl.ds(g * 16, 16)] * Q
                dg = didx[b][pl.ds(g * 16, 16)] * Q
                acc = jnp.zeros((16,), jnp.float32)
                for q in range(Q):
                    sv = plsc.load_gather(st, [sg + colrot[q]])
                    dv = plsc.load_gather(dt, [dg + colrot[q]])
                    s_lo = plsc.bitcast(lax.shift_left(sv, 16), jnp.float32)
                    d_lo = plsc.bitcast(lax.shift_left(dv, 16), jnp.float32)
                    s_hi = plsc.bitcast(
                        jnp.bitwise_and(sv, himask), jnp.float32)
                    d_hi = plsc.bitcast(
                        jnp.bitwise_and(dv, himask), jnp.float32)
                    acc = acc + s_lo * d_lo
                    acc = acc + s_hi * d_hi
                par[b][pl.ds(g * 16, 16)] = acc
                return c2

            lax.fori_loop(0, CE // 16, g_body, 0)
            pltpu.async_copy(
                par[b], par_hbm.at[pl.ds(pbase + c * CE, CE)], sem_p[b])
            fire_idx(b, c + 2)
        return carry

    lax.fori_loop(0, NCH // 2, s_body, 0)

    wait_idx(0)
    wait_idx(1)
    wait_par(0)
    wait_par(1)
    plsc.subcore_barrier()

    sbase = half + tid * SW
    for sub in range(SW // CR):
        off = sbase + sub * CR
        for t in range(NT):
            pltpu.async_copy(
                par_hbm.at[pl.ds(t * E + off, CR)], red[t], sem_r)
        for t in range(NT):
            pltpu.make_async_copy(
                par_hbm.at[pl.ds(0, CR)], red[t], sem_r).wait()

        def r_body(v, c2):
            acc = red[0][pl.ds(v * 16, 16)]
            for t in range(1, NT):
                acc = acc + red[t][pl.ds(v * 16, 16)]
            accv[pl.ds(v * 16, 16)] = acc
            return c2

        lax.fori_loop(0, CR // 16, r_body, 0)
        pltpu.sync_copy(accv, out_hbm.at[pl.ds(off, CR)])


def kernel(src_idx, dst_idx, src_feat, dst_feat):
    sf = jnp.transpose(src_feat).reshape(D * N)
    df = jnp.transpose(dst_feat).reshape(D * N)
    out, _ = _sddmm(src_idx.astype(jnp.int32), dst_idx.astype(jnp.int32),
                    sf, df)
    return out


# speedup vs baseline: 1.7786x; 1.1212x over previous
import functools

import jax
import jax.numpy as jnp
from jax import lax
from jax.experimental import pallas as pl
from jax.experimental.pallas import tpu as pltpu
from jax.experimental.pallas import tpu_sc as plsc

E = 320000
D = 128
N = 10000
NT = 16
Q = D // (2 * NT)
TW = N * Q
CE = 800
EH = E // 2
NCH = EH // CE
SW = EH // NT
CR = 400
NR = SW // CR
CN = 400
NC_T = N // CN

_mesh = plsc.VectorSubcoreMesh(core_axis_name="c", subcore_axis_name="s")


@functools.partial(
    pl.kernel,
    out_type=(
        jax.ShapeDtypeStruct((E,), jnp.float32),
        jax.ShapeDtypeStruct((NT * E,), jnp.float32),
    ),
    mesh=_mesh,
    scratch_types=[
        pltpu.VMEM((TW,), jnp.int32),
        pltpu.VMEM((TW,), jnp.int32),
        [pltpu.VMEM((CE,), jnp.int32)] * 2,
        [pltpu.VMEM((CE,), jnp.int32)] * 2,
        [pltpu.VMEM((CE,), jnp.float32)] * 2,
        [[pltpu.VMEM((CR,), jnp.float32)] * NT] * 2,
        [pltpu.VMEM((CR,), jnp.float32)] * 2,
        [[pltpu.VMEM((CN,), jnp.float32)] * 16] * 2,
        [pltpu.SemaphoreType.DMA] * 2,
        [pltpu.SemaphoreType.DMA] * 2,
        [pltpu.SemaphoreType.DMA] * 2,
        [pltpu.SemaphoreType.DMA] * 2,
        [pltpu.SemaphoreType.DMA] * 2,
    ],
    compiler_params=pltpu.CompilerParams(needs_layout_passes=False),
)
def _sddmm(src_idx_hbm, dst_idx_hbm, st_hbm, dt_hbm, out_hbm, par_hbm,
           st, dt, sidx, didx, par, red, accv, stg,
           sem_i, sem_p, sem_r, sem_o, sem_s):
    cid = lax.axis_index("c")
    tid = lax.axis_index("s")
    half = cid * EH
    pbase = tid * E + half
    last = NCH - 1

    def fire_idx(b, c):
        off = half + jnp.minimum(c, last) * CE
        pltpu.async_copy(src_idx_hbm.at[pl.ds(off, CE)], sidx[b], sem_i[b])
        pltpu.async_copy(dst_idx_hbm.at[pl.ds(off, CE)], didx[b], sem_i[b])

    def wait_idx(b):
        pltpu.make_async_copy(
            src_idx_hbm.at[pl.ds(half, CE)], sidx[b], sem_i[b]).wait()
        pltpu.make_async_copy(
            dst_idx_hbm.at[pl.ds(half, CE)], didx[b], sem_i[b]).wait()

    def wait_par(b):
        pltpu.make_async_copy(
            par[b], par_hbm.at[pl.ds(pbase, CE)], sem_p[b]).wait()

    fire_idx(0, 0)
    fire_idx(1, 1)

    lane = lax.iota(jnp.int32, 16)
    lane4 = lane * 4
    himask = jnp.full((16,), -65536, jnp.int32)

    rbase = 8 * tid * N

    def fire_tab(e, cb):
        nb = cb * CN
        for d8 in range(8):
            pltpu.async_copy(
                st_hbm.at[pl.ds(rbase + d8 * N + nb, CN)],
                stg[e][d8], sem_s[e])
            pltpu.async_copy(
                dt_hbm.at[pl.ds(rbase + d8 * N + nb, CN)],
                stg[e][8 + d8], sem_s[e])

    def wait_tab(e):
        for r in stg[e]:
            pltpu.make_async_copy(
                st_hbm.at[pl.ds(0, CN)], r, sem_s[e]).wait()

    def rnd(v):
        b = plsc.bitcast(v, jnp.int32)
        return b + 32767 + jnp.bitwise_and(
            lax.shift_right_logical(b, 16), 1)

    def repack(e, cb):
        nb = cb * CN

        def n_body(g, c2):
            idx0 = (nb + g * 16) * 4 + lane4
            for base_r, dst_ref in ((0, st), (8, dt)):
                for q in range(Q):
                    ev = rnd(stg[e][base_r + 2 * q][pl.ds(g * 16, 16)])
                    ov = rnd(stg[e][base_r + 2 * q + 1][pl.ds(g * 16, 16)])
                    pk16 = jnp.bitwise_or(
                        lax.shift_right_logical(ev, 16),
                        jnp.bitwise_and(ov, himask))
                    plsc.store_scatter(dst_ref, [idx0 + q], pk16)
            return c2

        lax.fori_loop(0, CN // 16, n_body, 0)

    fire_tab(0, 0)

    def tb_body(s, carry):
        for b in range(2):
            cb = s * 2 + b
            fire_tab(1 - b, cb + 1)
            wait_tab(b)
            repack(b, cb)
        return carry

    lax.fori_loop(0, (NC_T - 1) // 2, tb_body, 0)
    wait_tab(0)
    repack(0, NC_T - 1)

    colrot = tuple(jnp.bitwise_and(lane + q, Q - 1) for q in range(Q))

    def s_body(s, carry):
        for b in range(2):
            c = s * 2 + b
            wait_idx(b)

            @pl.when(s > 0)
            def _():
                wait_par(b)

            def g_body(g, c2):
                sg = sidx[b][pl.ds(g * 16, 16)] * Q
                dg = didx[b][pl.ds(g * 16, 16)] * Q
                acc = jnp.zeros((16,), jnp.float32)
                for q in range(Q):
                    sv = plsc.load_gather(st, [sg + colrot[q]])
                    dv = plsc.load_gather(dt, [dg + colrot[q]])
                    s_lo = plsc.bitcast(lax.shift_left(sv, 16), jnp.float32)
                    d_lo = plsc.bitcast(lax.shift_left(dv, 16), jnp.float32)
                    s_hi = plsc.bitcast(
                        jnp.bitwise_and(sv, himask), jnp.float32)
                    d_hi = plsc.bitcast(
                        jnp.bitwise_and(dv, himask), jnp.float32)
                    acc = acc + s_lo * d_lo
                    acc = acc + s_hi * d_hi
                par[b][pl.ds(g * 16, 16)] = acc
                return c2

            lax.fori_loop(0, CE // 16, g_body, 0)
            pltpu.async_copy(
                par[b], par_hbm.at[pl.ds(pbase + c * CE, CE)], sem_p[b])
            fire_idx(b, c + 2)
        return carry

    lax.fori_loop(0, NCH // 2, s_body, 0)

    wait_idx(0)
    wait_idx(1)
    wait_par(0)
    wait_par(1)
    plsc.subcore_barrier()

    sbase = half + tid * SW

    def fire_red(e, r):
        off = sbase + r * CR
        for t in range(NT):
            pltpu.async_copy(
                par_hbm.at[pl.ds(t * E + off, CR)], red[e][t], sem_r[e])

    def wait_red(e):
        for t in range(NT):
            pltpu.make_async_copy(
                par_hbm.at[pl.ds(0, CR)], red[e][t], sem_r[e]).wait()

    def wait_out(e):
        pltpu.make_async_copy(
            accv[e], out_hbm.at[pl.ds(sbase, CR)], sem_o[e]).wait()

    def reduce_round(e, r):
        def r_body(v, c2):
            acc = red[e][0][pl.ds(v * 16, 16)]
            for t in range(1, NT):
                acc = acc + red[e][t][pl.ds(v * 16, 16)]
            accv[e][pl.ds(v * 16, 16)] = acc
            return c2

        lax.fori_loop(0, CR // 16, r_body, 0)
        pltpu.async_copy(
            accv[e], out_hbm.at[pl.ds(sbase + r * CR, CR)], sem_o[e])

    fire_red(0, 0)

    def p2_body(s, carry):
        for b in range(2):
            r = s * 2 + b
            fire_red(1 - b, r + 1)
            wait_red(b)

            @pl.when(s > 0)
            def _():
                wait_out(b)

            reduce_round(b, r)
        return carry

    lax.fori_loop(0, (NR - 1) // 2, p2_body, 0)
    wait_red(0)
    wait_out(0)
    reduce_round(0, NR - 1)
    wait_out(0)
    wait_out(1)


def kernel(src_idx, dst_idx, src_feat, dst_feat):
    sf = jnp.transpose(src_feat).reshape(D * N)
    df = jnp.transpose(dst_feat).reshape(D * N)
    out, _ = _sddmm(src_idx.astype(jnp.int32), dst_idx.astype(jnp.int32),
                    sf, df)
    return out


# speedup vs baseline: 1.8069x; 1.0159x over previous
import functools

import jax
import jax.numpy as jnp
from jax import lax
from jax.experimental import pallas as pl
from jax.experimental.pallas import tpu as pltpu
from jax.experimental.pallas import tpu_sc as plsc

E = 320000
D = 128
N = 10000
NT = 16
Q = D // (2 * NT)
TW = N * Q
CE = 1600
EH = E // 2
NCH = EH // CE
SW = EH // NT
CR = 400
NR = SW // CR
CN = 400
NC_T = N // CN

_mesh = plsc.VectorSubcoreMesh(core_axis_name="c", subcore_axis_name="s")


@functools.partial(
    pl.kernel,
    out_type=(
        jax.ShapeDtypeStruct((E,), jnp.float32),
        jax.ShapeDtypeStruct((NT * E,), jnp.float32),
    ),
    mesh=_mesh,
    scratch_types=[
        pltpu.VMEM((TW,), jnp.int32),
        pltpu.VMEM((TW,), jnp.int32),
        [pltpu.VMEM((CE,), jnp.int32)] * 2,
        [pltpu.VMEM((CE,), jnp.int32)] * 2,
        [pltpu.VMEM((CE,), jnp.float32)] * 2,
        [[pltpu.VMEM((CR,), jnp.float32)] * NT] * 2,
        [pltpu.VMEM((CR,), jnp.float32)] * 2,
        [[pltpu.VMEM((CN,), jnp.float32)] * 16] * 2,
        [pltpu.SemaphoreType.DMA] * 2,
        [pltpu.SemaphoreType.DMA] * 2,
        [pltpu.SemaphoreType.DMA] * 2,
        [pltpu.SemaphoreType.DMA] * 2,
        [pltpu.SemaphoreType.DMA] * 2,
    ],
    compiler_params=pltpu.CompilerParams(needs_layout_passes=False),
)
def _sddmm(src_idx_hbm, dst_idx_hbm, st_hbm, dt_hbm, out_hbm, par_hbm,
           st, dt, sidx, didx, par, red, accv, stg,
           sem_i, sem_p, sem_r, sem_o, sem_s):
    cid = lax.axis_index("c")
    tid = lax.axis_index("s")
    half = cid * EH
    pbase = tid * E + half
    last = NCH - 1

    def fire_idx(b, c):
        off = half + jnp.minimum(c, last) * CE
        pltpu.async_copy(src_idx_hbm.at[pl.ds(off, CE)], sidx[b], sem_i[b])
        pltpu.async_copy(dst_idx_hbm.at[pl.ds(off, CE)], didx[b], sem_i[b])

    def wait_idx(b):
        pltpu.make_async_copy(
            src_idx_hbm.at[pl.ds(half, CE)], sidx[b], sem_i[b]).wait()
        pltpu.make_async_copy(
            dst_idx_hbm.at[pl.ds(half, CE)], didx[b], sem_i[b]).wait()

    def wait_par(b):
        pltpu.make_async_copy(
            par[b], par_hbm.at[pl.ds(pbase, CE)], sem_p[b]).wait()

    fire_idx(0, 0)
    fire_idx(1, 1)

    lane = lax.iota(jnp.int32, 16)
    lane4 = lane * 4
    himask = jnp.full((16,), -65536, jnp.int32)

    rbase = 8 * tid * N

    def fire_tab(e, cb):
        nb = cb * CN
        for d8 in range(8):
            pltpu.async_copy(
                st_hbm.at[pl.ds(rbase + d8 * N + nb, CN)],
                stg[e][d8], sem_s[e])
            pltpu.async_copy(
                dt_hbm.at[pl.ds(rbase + d8 * N + nb, CN)],
                stg[e][8 + d8], sem_s[e])

    def wait_tab(e):
        for r in stg[e]:
            pltpu.make_async_copy(
                st_hbm.at[pl.ds(0, CN)], r, sem_s[e]).wait()

    def rnd(v):
        b = plsc.bitcast(v, jnp.int32)
        return b + 32767 + jnp.bitwise_and(
            lax.shift_right_logical(b, 16), 1)

    def repack(e, cb):
        nb = cb * CN

        def n_body(g, c2):
            idx0 = (nb + g * 16) * 4 + lane4
            for base_r, dst_ref in ((0, st), (8, dt)):
                for q in range(Q):
                    ev = rnd(stg[e][base_r + 2 * q][pl.ds(g * 16, 16)])
                    ov = rnd(stg[e][base_r + 2 * q + 1][pl.ds(g * 16, 16)])
                    pk16 = jnp.bitwise_or(
                        lax.shift_right_logical(ev, 16),
                        jnp.bitwise_and(ov, himask))
                    plsc.store_scatter(dst_ref, [idx0 + q], pk16)
            return c2

        lax.fori_loop(0, CN // 16, n_body, 0)

    fire_tab(0, 0)

    def tb_body(s, carry):
        for b in range(2):
            cb = s * 2 + b
            fire_tab(1 - b, cb + 1)
            wait_tab(b)
            repack(b, cb)
        return carry

    lax.fori_loop(0, (NC_T - 1) // 2, tb_body, 0)
    wait_tab(0)
    repack(0, NC_T - 1)

    colrot = tuple(jnp.bitwise_and(lane + q, Q - 1) for q in range(Q))

    def s_body(s, carry):
        for b in range(2):
            c = s * 2 + b
            wait_idx(b)

            @pl.when(s > 0)
            def _():
                wait_par(b)

            def g_body(g, c2):
                sg = sidx[b][pl.ds(g * 16, 16)] * Q
                dg = didx[b][pl.ds(g * 16, 16)] * Q
                acc = jnp.zeros((16,), jnp.float32)
                for q in range(Q):
                    sv = plsc.load_gather(st, [sg + colrot[q]])
                    dv = plsc.load_gather(dt, [dg + colrot[q]])
                    s_lo = plsc.bitcast(lax.shift_left(sv, 16), jnp.float32)
                    d_lo = plsc.bitcast(lax.shift_left(dv, 16), jnp.float32)
                    s_hi = plsc.bitcast(
                        jnp.bitwise_and(sv, himask), jnp.float32)
                    d_hi = plsc.bitcast(
                        jnp.bitwise_and(dv, himask), jnp.float32)
                    acc = acc + s_lo * d_lo
                    acc = acc + s_hi * d_hi
                par[b][pl.ds(g * 16, 16)] = acc
                return c2

            lax.fori_loop(0, CE // 16, g_body, 0)
            pltpu.async_copy(
                par[b], par_hbm.at[pl.ds(pbase + c * CE, CE)], sem_p[b])
            fire_idx(b, c + 2)
        return carry

    lax.fori_loop(0, NCH // 2, s_body, 0)

    wait_idx(0)
    wait_idx(1)
    wait_par(0)
    wait_par(1)
    plsc.subcore_barrier()

    sbase = half + tid * SW

    def fire_red(e, r):
        off = sbase + r * CR
        for t in range(NT):
            pltpu.async_copy(
                par_hbm.at[pl.ds(t * E + off, CR)], red[e][t], sem_r[e])

    def wait_red(e):
        for t in range(NT):
            pltpu.make_async_copy(
                par_hbm.at[pl.ds(0, CR)], red[e][t], sem_r[e]).wait()

    def wait_out(e):
        pltpu.make_async_copy(
            accv[e], out_hbm.at[pl.ds(sbase, CR)], sem_o[e]).wait()

    def reduce_round(e, r):
        def r_body(v, c2):
            acc = red[e][0][pl.ds(v * 16, 16)]
            for t in range(1, NT):
                acc = acc + red[e][t][pl.ds(v * 16, 16)]
            accv[e][pl.ds(v * 16, 16)] = acc
            return c2

        lax.fori_loop(0, CR // 16, r_body, 0)
        pltpu.async_copy(
            accv[e], out_hbm.at[pl.ds(sbase + r * CR, CR)], sem_o[e])

    fire_red(0, 0)

    def p2_body(s, carry):
        for b in range(2):
            r = s * 2 + b
            fire_red(1 - b, r + 1)
            wait_red(b)

            @pl.when(s > 0)
            def _():
                wait_out(b)

            reduce_round(b, r)
        return carry

    lax.fori_loop(0, (NR - 1) // 2, p2_body, 0)
    wait_red(0)
    wait_out(0)
    reduce_round(0, NR - 1)
    wait_out(0)
    wait_out(1)


def kernel(src_idx, dst_idx, src_feat, dst_feat):
    sf = jnp.transpose(src_feat).reshape(D * N)
    df = jnp.transpose(dst_feat).reshape(D * N)
    out, _ = _sddmm(src_idx.astype(jnp.int32), dst_idx.astype(jnp.int32),
                    sf, df)
    return out
